# Initial kernel scaffold; baseline (speedup 1.0000x reference)
import jax
import jax.numpy as jnp
from jax.experimental import pallas as pl


def kernel(x, edge_index, W1, b1, W2, b2, Wlin, blin):
    raise NotImplementedError("write your pallas kernel here")


# speedup vs baseline: 6.4621x; 6.4621x over previous
import functools

import jax
import jax.numpy as jnp
from jax import lax
from jax.experimental import pallas as pl
from jax.experimental.pallas import tpu as pltpu
from jax.experimental.pallas import tpu_sc as plsc

NC, NS = 2, 16
CH = 16
D = 128


def _sc_mesh():
    return plsc.VectorSubcoreMesh(
        core_axis_name="c", subcore_axis_name="s", num_cores=NC, num_subcores=NS
    )


def _deg_call(E, N_pad):
    e_per_tile = E // (NC * NS)
    n_chunks = e_per_tile // CH
    rpt = N_pad // NS

    @functools.partial(
        pl.kernel,
        out_type=jax.ShapeDtypeStruct((NC, N_pad, D), jnp.float32),
        mesh=_sc_mesh(),
        scratch_types=[
            pltpu.VMEM((e_per_tile,), jnp.int32),
            pltpu.VMEM((CH, D), jnp.float32),
            pltpu.VMEM_SHARED((N_pad, D), jnp.float32),
            pltpu.SemaphoreType.DMA,
        ],
    )
    def deg_k(dst_hbm, ones_hbm, zeros_hbm, out_hbm, idx_v, ones_v, acc_s, sem):
        cid = lax.axis_index("c")
        sid = lax.axis_index("s")
        wid = cid * NS + sid
        r0 = sid * rpt
        pltpu.sync_copy(zeros_hbm.at[pl.ds(r0, rpt)], acc_s.at[pl.ds(r0, rpt)])
        pltpu.sync_copy(dst_hbm.at[pl.ds(wid * e_per_tile, e_per_tile)], idx_v)
        pltpu.sync_copy(ones_hbm, ones_v)
        plsc.subcore_barrier()

        def body(j, c):
            iv = idx_v[pl.ds(j * CH, CH)]
            pltpu.async_copy(ones_v, acc_s.at[iv], sem, add=True).wait()
            return c

        lax.fori_loop(0, n_chunks, body, 0)
        plsc.subcore_barrier()
        pltpu.sync_copy(acc_s.at[pl.ds(r0, rpt)], out_hbm.at[cid, pl.ds(r0, rpt)])

    return deg_k


def _agg_call(E, N_pad, feature_split):
    e_per_tile = E // NS if feature_split else E // (NC * NS)
    n_chunks = e_per_tile // CH
    rpt = N_pad // NS
    n_tables = 2 if feature_split else 1

    @functools.partial(
        pl.kernel,
        out_type=jax.ShapeDtypeStruct((NC, N_pad, D), jnp.float32),
        mesh=_sc_mesh(),
        scratch_types=[
            pltpu.VMEM((e_per_tile,), jnp.int32),
            pltpu.VMEM((e_per_tile,), jnp.int32),
            pltpu.VMEM((CH, D), jnp.float32),
            pltpu.VMEM_SHARED((N_pad, D), jnp.float32),
            pltpu.SemaphoreType.DMA,
            pltpu.SemaphoreType.DMA,
        ],
    )
    def agg_k(*args):
        tables = args[:n_tables]
        src_hbm, dst_hbm, zeros_hbm, out_hbm = args[n_tables : n_tables + 4]
        src_v, dst_v, rows_v, acc_s, gsem, ssem = args[n_tables + 4 :]
        cid = lax.axis_index("c")
        sid = lax.axis_index("s")
        r0 = sid * rpt
        pltpu.sync_copy(zeros_hbm.at[pl.ds(r0, rpt)], acc_s.at[pl.ds(r0, rpt)])
        base = (sid if feature_split else cid * NS + sid) * e_per_tile
        pltpu.sync_copy(src_hbm.at[pl.ds(base, e_per_tile)], src_v)
        pltpu.sync_copy(dst_hbm.at[pl.ds(base, e_per_tile)], dst_v)
        plsc.subcore_barrier()

        def run(table):
            def body(j, c):
                sv = src_v[pl.ds(j * CH, CH)]
                pltpu.async_copy(table.at[sv], rows_v, gsem).wait()
                dv = dst_v[pl.ds(j * CH, CH)]
                pltpu.async_copy(rows_v, acc_s.at[dv], ssem, add=True).wait()
                return c

            lax.fori_loop(0, n_chunks, body, 0)

        if feature_split:
            pl.when(cid == 0)(lambda: run(tables[0]))
            pl.when(cid == 1)(lambda: run(tables[1]))
        else:
            run(tables[0])
        plsc.subcore_barrier()
        pltpu.sync_copy(acc_s.at[pl.ds(r0, rpt)], out_hbm.at[cid, pl.ds(r0, rpt)])

    return agg_k


def _dot(a, b):
    return jax.lax.dot_general(
        a, b, (((1,), (0,)), ((), ())),
        preferred_element_type=jnp.float32,
        precision=jax.lax.Precision.HIGHEST,
    )


def _prep1(x, W1a, W1b, degh, BLK=1000):
    N, K = x.shape

    def body(x_ref, wa_ref, wb_ref, dg_ref, lo_ref, hi_ref, dv_ref):
        deg = dg_ref[0, :, 0] + dg_ref[1, :, 0] + 1.0
        dinv = lax.rsqrt(deg)
        xs = x_ref[...] * dinv[:, None]
        lo_ref[...] = _dot(xs, wa_ref[...])
        hi_ref[...] = _dot(xs, wb_ref[...])
        dv_ref[...] = jnp.broadcast_to(dinv[:, None], (BLK, D))

    return pl.pallas_call(
        body,
        grid=(N // BLK,),
        in_specs=[
            pl.BlockSpec((BLK, K), lambda i: (i, 0)),
            pl.BlockSpec((K, D), lambda i: (0, 0)),
            pl.BlockSpec((K, D), lambda i: (0, 0)),
            pl.BlockSpec((NC, BLK, D), lambda i: (0, i, 0)),
        ],
        out_specs=[pl.BlockSpec((BLK, D), lambda i: (i, 0))] * 3,
        out_shape=[jax.ShapeDtypeStruct((N, D), jnp.float32)] * 3,
    )(x, W1a, W1b, degh)


def _mid(agg1, hs_lo, hs_hi, dinvb, b1a, b1b, W2a, W2b, BLK=1000):
    N = hs_lo.shape[0]

    def body(ag_ref, lo_ref, hi_ref, dv_ref, ba_ref, bb_ref, wa_ref, wb_ref, o_ref):
        dinv = dv_ref[...]
        ta = jax.nn.relu(dinv * (ag_ref[0] + lo_ref[...]) + ba_ref[...]) * dinv
        tb = jax.nn.relu(dinv * (ag_ref[1] + hi_ref[...]) + bb_ref[...]) * dinv
        o_ref[...] = _dot(ta, wa_ref[...]) + _dot(tb, wb_ref[...])

    return pl.pallas_call(
        body,
        grid=(N // BLK,),
        in_specs=[
            pl.BlockSpec((NC, BLK, D), lambda i: (0, i, 0)),
            pl.BlockSpec((BLK, D), lambda i: (i, 0)),
            pl.BlockSpec((BLK, D), lambda i: (i, 0)),
            pl.BlockSpec((BLK, D), lambda i: (i, 0)),
            pl.BlockSpec((1, D), lambda i: (0, 0)),
            pl.BlockSpec((1, D), lambda i: (0, 0)),
            pl.BlockSpec((D, D), lambda i: (0, 0)),
            pl.BlockSpec((D, D), lambda i: (0, 0)),
        ],
        out_specs=pl.BlockSpec((BLK, D), lambda i: (i, 0)),
        out_shape=jax.ShapeDtypeStruct((N, D), jnp.float32),
    )(agg1, hs_lo, hs_hi, dinvb, b1a, b1b, W2a, W2b)


def _fin(agg2, hs2, dinvb, b2r, Wlin, blinr, BLK=1000):
    N = hs2.shape[0]

    def body(ag_ref, h_ref, dv_ref, b_ref, w_ref, bl_ref, o_ref):
        dinv = dv_ref[...]
        agg = ag_ref[0] + ag_ref[1]
        t = jax.nn.relu(dinv * (agg + h_ref[...]) + b_ref[...])
        o_ref[...] = _dot(t, w_ref[...]) + bl_ref[...]

    return pl.pallas_call(
        body,
        grid=(N // BLK,),
        in_specs=[
            pl.BlockSpec((NC, BLK, D), lambda i: (0, i, 0)),
            pl.BlockSpec((BLK, D), lambda i: (i, 0)),
            pl.BlockSpec((BLK, D), lambda i: (i, 0)),
            pl.BlockSpec((1, D), lambda i: (0, 0)),
            pl.BlockSpec((D, D), lambda i: (0, 0)),
            pl.BlockSpec((1, D), lambda i: (0, 0)),
        ],
        out_specs=pl.BlockSpec((BLK, D), lambda i: (i, 0)),
        out_shape=jax.ShapeDtypeStruct((N, D), jnp.float32),
    )(agg2, hs2, dinvb, b2r, Wlin, blinr)


def kernel(x, edge_index, W1, b1, W2, b2, Wlin, blin):
    N = x.shape[0]
    E = edge_index.shape[1]
    N_pad = -(-N // (NS * 8)) * (NS * 8)
    ei = edge_index.astype(jnp.int32)
    src, dst = ei[0], ei[1]
    zeros = jnp.zeros((N_pad, D), jnp.float32)
    ones = jnp.ones((CH, D), jnp.float32)

    degh = _deg_call(E, N_pad)(dst, ones, zeros)
    hs_lo, hs_hi, dinvb = _prep1(x, W1[:, :D], W1[:, D:], degh)
    agg1 = _agg_call(E, N_pad, True)(hs_lo, hs_hi, src, dst, zeros)
    hs2 = _mid(
        agg1, hs_lo, hs_hi, dinvb,
        b1[:D].reshape(1, D), b1[D:].reshape(1, D), W2[:D], W2[D:],
    )
    agg2 = _agg_call(E, N_pad, False)(hs2, src, dst, zeros)
    return _fin(agg2, hs2, dinvb, b2.reshape(1, D), Wlin, blin.reshape(1, D))


# speedup vs baseline: 11.4596x; 1.7734x over previous
import functools

import jax
import jax.numpy as jnp
from jax import lax
from jax.experimental import pallas as pl
from jax.experimental.pallas import tpu as pltpu
from jax.experimental.pallas import tpu_sc as plsc

NC, NS = 2, 16
CH = 16
D = 128
GDEG = 5


def _sc_mesh():
    return plsc.VectorSubcoreMesh(
        core_axis_name="c", subcore_axis_name="s", num_cores=NC, num_subcores=NS
    )


def _deg_call(E, N_pad):
    e_per_tile = E // (NC * NS)
    n_chunks = e_per_tile // CH
    rpt = N_pad // NS

    @functools.partial(
        pl.kernel,
        out_type=jax.ShapeDtypeStruct((NC, N_pad, D), jnp.float32),
        mesh=_sc_mesh(),
        scratch_types=[
            pltpu.VMEM((e_per_tile,), jnp.int32),
            pltpu.VMEM((CH, D), jnp.float32),
            pltpu.VMEM_SHARED((N_pad, D), jnp.float32),
            pltpu.SemaphoreType.DMA,
        ],
    )
    def deg_k(dst_hbm, ones_hbm, zeros_hbm, out_hbm, idx_v, ones_v, acc_s, sem):
        cid = lax.axis_index("c")
        sid = lax.axis_index("s")
        wid = cid * NS + sid
        r0 = sid * rpt
        pltpu.sync_copy(zeros_hbm.at[pl.ds(r0, rpt)], acc_s.at[pl.ds(r0, rpt)])
        pltpu.sync_copy(dst_hbm.at[pl.ds(wid * e_per_tile, e_per_tile)], idx_v)
        pltpu.sync_copy(ones_hbm, ones_v)
        plsc.subcore_barrier()

        def body(g, c):
            descs = []
            for k in range(GDEG):
                iv = idx_v[pl.ds((g * GDEG + k) * CH, CH)]
                descs.append(
                    pltpu.async_copy(ones_v, acc_s.at[iv], sem, add=True)
                )
            for d in descs:
                d.wait()
            return c

        lax.fori_loop(0, n_chunks // GDEG, body, 0)
        plsc.subcore_barrier()
        pltpu.sync_copy(acc_s.at[pl.ds(r0, rpt)], out_hbm.at[cid, pl.ds(r0, rpt)])

    return deg_k


def _agg_call(E, N_pad, feature_split):
    e_per_tile = E // NS if feature_split else E // (NC * NS)
    n_chunks = e_per_tile // CH
    G = 5
    rpt = N_pad // NS
    n_tables = 2 if feature_split else 1

    @functools.partial(
        pl.kernel,
        out_type=jax.ShapeDtypeStruct((NC, N_pad, D), jnp.float32),
        mesh=_sc_mesh(),
        scratch_types=[
            pltpu.VMEM((G * CH,), jnp.int32),
            pltpu.VMEM((G * CH,), jnp.int32),
            pltpu.VMEM((G * CH, D), jnp.float32),
            pltpu.VMEM_SHARED((N_pad, D), jnp.float32),
            pltpu.SemaphoreType.DMA,
            pltpu.SemaphoreType.DMA,
        ],
    )
    def agg_k(*args):
        tables = args[:n_tables]
        src_hbm, dst_hbm, zeros_hbm, out_hbm = args[n_tables : n_tables + 4]
        src_v, dst_v, rows_v, acc_s, gsem, ssem = args[n_tables + 4 :]
        cid = lax.axis_index("c")
        sid = lax.axis_index("s")
        r0 = sid * rpt
        pltpu.sync_copy(zeros_hbm.at[pl.ds(r0, rpt)], acc_s.at[pl.ds(r0, rpt)])
        base = (sid if feature_split else cid * NS + sid) * e_per_tile
        plsc.subcore_barrier()

        def run(table):
            def body(g, c):
                e0 = base + g * G * CH
                pltpu.sync_copy(src_hbm.at[pl.ds(e0, G * CH)], src_v)
                pltpu.sync_copy(dst_hbm.at[pl.ds(e0, G * CH)], dst_v)
                gd = []
                for k in range(G):
                    sv = src_v[pl.ds(k * CH, CH)]
                    gd.append(
                        pltpu.async_copy(
                            table.at[sv], rows_v.at[pl.ds(k * CH, CH)], gsem
                        )
                    )
                sd = []
                for k in range(G):
                    gd[k].wait()
                    dv = dst_v[pl.ds(k * CH, CH)]
                    sd.append(
                        pltpu.async_copy(
                            rows_v.at[pl.ds(k * CH, CH)],
                            acc_s.at[dv],
                            ssem,
                            add=True,
                        )
                    )
                for d in sd:
                    d.wait()
                return c

            lax.fori_loop(0, n_chunks // G, body, 0)

        if feature_split:
            pl.when(cid == 0)(lambda: run(tables[0]))
            pl.when(cid == 1)(lambda: run(tables[1]))
        else:
            run(tables[0])
        plsc.subcore_barrier()
        pltpu.sync_copy(acc_s.at[pl.ds(r0, rpt)], out_hbm.at[cid, pl.ds(r0, rpt)])

    return agg_k


def _dot(a, b):
    return jax.lax.dot_general(
        a, b, (((1,), (0,)), ((), ())),
        preferred_element_type=jnp.float32,
        precision=jax.lax.Precision.HIGHEST,
    )


def _prep1(x, W1a, W1b, degh, BLK=1000):
    N, K = x.shape

    def body(x_ref, wa_ref, wb_ref, dg_ref, lo_ref, hi_ref, dv_ref):
        deg = dg_ref[0, :, 0] + dg_ref[1, :, 0] + 1.0
        dinv = lax.rsqrt(deg)
        xs = x_ref[...] * dinv[:, None]
        lo_ref[...] = _dot(xs, wa_ref[...])
        hi_ref[...] = _dot(xs, wb_ref[...])
        dv_ref[...] = jnp.broadcast_to(dinv[:, None], (BLK, D))

    return pl.pallas_call(
        body,
        grid=(N // BLK,),
        in_specs=[
            pl.BlockSpec((BLK, K), lambda i: (i, 0)),
            pl.BlockSpec((K, D), lambda i: (0, 0)),
            pl.BlockSpec((K, D), lambda i: (0, 0)),
            pl.BlockSpec((NC, BLK, D), lambda i: (0, i, 0)),
        ],
        out_specs=[pl.BlockSpec((BLK, D), lambda i: (i, 0))] * 3,
        out_shape=[jax.ShapeDtypeStruct((N, D), jnp.float32)] * 3,
    )(x, W1a, W1b, degh)


def _mid(agg1, hs_lo, hs_hi, dinvb, b1a, b1b, W2a, W2b, BLK=1000):
    N = hs_lo.shape[0]

    def body(ag_ref, lo_ref, hi_ref, dv_ref, ba_ref, bb_ref, wa_ref, wb_ref, o_ref):
        dinv = dv_ref[...]
        ta = jax.nn.relu(dinv * (ag_ref[0] + lo_ref[...]) + ba_ref[...]) * dinv
        tb = jax.nn.relu(dinv * (ag_ref[1] + hi_ref[...]) + bb_ref[...]) * dinv
        o_ref[...] = _dot(ta, wa_ref[...]) + _dot(tb, wb_ref[...])

    return pl.pallas_call(
        body,
        grid=(N // BLK,),
        in_specs=[
            pl.BlockSpec((NC, BLK, D), lambda i: (0, i, 0)),
            pl.BlockSpec((BLK, D), lambda i: (i, 0)),
            pl.BlockSpec((BLK, D), lambda i: (i, 0)),
            pl.BlockSpec((BLK, D), lambda i: (i, 0)),
            pl.BlockSpec((1, D), lambda i: (0, 0)),
            pl.BlockSpec((1, D), lambda i: (0, 0)),
            pl.BlockSpec((D, D), lambda i: (0, 0)),
            pl.BlockSpec((D, D), lambda i: (0, 0)),
        ],
        out_specs=pl.BlockSpec((BLK, D), lambda i: (i, 0)),
        out_shape=jax.ShapeDtypeStruct((N, D), jnp.float32),
    )(agg1, hs_lo, hs_hi, dinvb, b1a, b1b, W2a, W2b)


def _fin(agg2, hs2, dinvb, b2r, Wlin, blinr, BLK=1000):
    N = hs2.shape[0]

    def body(ag_ref, h_ref, dv_ref, b_ref, w_ref, bl_ref, o_ref):
        dinv = dv_ref[...]
        agg = ag_ref[0] + ag_ref[1]
        t = jax.nn.relu(dinv * (agg + h_ref[...]) + b_ref[...])
        o_ref[...] = _dot(t, w_ref[...]) + bl_ref[...]

    return pl.pallas_call(
        body,
        grid=(N // BLK,),
        in_specs=[
            pl.BlockSpec((NC, BLK, D), lambda i: (0, i, 0)),
            pl.BlockSpec((BLK, D), lambda i: (i, 0)),
            pl.BlockSpec((BLK, D), lambda i: (i, 0)),
            pl.BlockSpec((1, D), lambda i: (0, 0)),
            pl.BlockSpec((D, D), lambda i: (0, 0)),
            pl.BlockSpec((1, D), lambda i: (0, 0)),
        ],
        out_specs=pl.BlockSpec((BLK, D), lambda i: (i, 0)),
        out_shape=jax.ShapeDtypeStruct((N, D), jnp.float32),
    )(agg2, hs2, dinvb, b2r, Wlin, blinr)


def kernel(x, edge_index, W1, b1, W2, b2, Wlin, blin):
    N = x.shape[0]
    E = edge_index.shape[1]
    N_pad = -(-N // (NS * 8)) * (NS * 8)
    ei = edge_index.astype(jnp.int32)
    src, dst = ei[0], ei[1]
    zeros = jnp.zeros((N_pad, D), jnp.float32)
    ones = jnp.ones((CH, D), jnp.float32)

    degh = _deg_call(E, N_pad)(dst, ones, zeros)
    hs_lo, hs_hi, dinvb = _prep1(x, W1[:, :D], W1[:, D:], degh)
    agg1 = _agg_call(E, N_pad, True)(hs_lo, hs_hi, src, dst, zeros)
    hs2 = _mid(
        agg1, hs_lo, hs_hi, dinvb,
        b1[:D].reshape(1, D), b1[D:].reshape(1, D), W2[:D], W2[D:],
    )
    agg2 = _agg_call(E, N_pad, False)(hs2, src, dst, zeros)
    return _fin(agg2, hs2, dinvb, b2.reshape(1, D), Wlin, blin.reshape(1, D))


# speedup vs baseline: 16.0601x; 1.4015x over previous
import functools

import jax
import jax.numpy as jnp
from jax import lax
from jax.experimental import pallas as pl
from jax.experimental.pallas import tpu as pltpu
from jax.experimental.pallas import tpu_sc as plsc

NC, NS = 2, 16
CH = 16
D = 128
GDEG = 25


def _sc_mesh():
    return plsc.VectorSubcoreMesh(
        core_axis_name="c", subcore_axis_name="s", num_cores=NC, num_subcores=NS
    )


def _deg_call(E, N_pad):
    e_per_tile = E // (NC * NS)
    n_chunks = e_per_tile // CH
    rpt = N_pad // NS

    @functools.partial(
        pl.kernel,
        out_type=jax.ShapeDtypeStruct((NC, N_pad, D), jnp.float32),
        mesh=_sc_mesh(),
        scratch_types=[
            pltpu.VMEM((e_per_tile,), jnp.int32),
            pltpu.VMEM((CH, D), jnp.float32),
            pltpu.VMEM_SHARED((N_pad, D), jnp.float32),
            pltpu.SemaphoreType.DMA,
        ],
    )
    def deg_k(dst_hbm, ones_hbm, zeros_hbm, out_hbm, idx_v, ones_v, acc_s, sem):
        cid = lax.axis_index("c")
        sid = lax.axis_index("s")
        wid = cid * NS + sid
        r0 = sid * rpt
        pltpu.sync_copy(zeros_hbm.at[pl.ds(r0, rpt)], acc_s.at[pl.ds(r0, rpt)])
        pltpu.sync_copy(dst_hbm.at[pl.ds(wid * e_per_tile, e_per_tile)], idx_v)
        pltpu.sync_copy(ones_hbm, ones_v)
        plsc.subcore_barrier()

        def body(g, c):
            descs = []
            for k in range(GDEG):
                iv = idx_v[pl.ds((g * GDEG + k) * CH, CH)]
                descs.append(
                    pltpu.async_copy(ones_v, acc_s.at[iv], sem, add=True)
                )
            for d in descs:
                d.wait()
            return c

        lax.fori_loop(0, n_chunks // GDEG, body, 0)
        plsc.subcore_barrier()
        pltpu.sync_copy(acc_s.at[pl.ds(r0, rpt)], out_hbm.at[cid, pl.ds(r0, rpt)])

    return deg_k


def _agg_call(E, N_pad, feature_split):
    e_per_tile = E // NS if feature_split else E // (NC * NS)
    n_chunks = e_per_tile // CH
    G = 10 if feature_split else 5
    rpt = N_pad // NS
    n_tables = 2 if feature_split else 1

    @functools.partial(
        pl.kernel,
        out_type=jax.ShapeDtypeStruct((NC, N_pad, D), jnp.float32),
        mesh=_sc_mesh(),
        scratch_types=[
            pltpu.VMEM((2, G * CH), jnp.int32),
            pltpu.VMEM((G * CH, D), jnp.float32),
            pltpu.VMEM_SHARED((N_pad, D), jnp.float32),
            pltpu.SemaphoreType.DMA,
            pltpu.SemaphoreType.DMA,
        ],
    )
    def agg_k(*args):
        tables = args[:n_tables]
        pk_hbm, zeros_hbm, out_hbm = args[n_tables : n_tables + 3]
        idx_v, rows_v, acc_s, gsem, ssem = args[n_tables + 3 :]
        cid = lax.axis_index("c")
        sid = lax.axis_index("s")
        r0 = sid * rpt
        pltpu.sync_copy(zeros_hbm.at[pl.ds(r0, rpt)], acc_s.at[pl.ds(r0, rpt)])
        tid = sid if feature_split else cid * NS + sid
        plsc.subcore_barrier()

        def run(table):
            def body(g, c):
                pltpu.sync_copy(pk_hbm.at[tid, g], idx_v)
                gd = []
                for k in range(G):
                    sv = idx_v[0, pl.ds(k * CH, CH)]
                    gd.append(
                        pltpu.async_copy(
                            table.at[sv], rows_v.at[pl.ds(k * CH, CH)], gsem
                        )
                    )
                sd = []
                for k in range(G):
                    gd[k].wait()
                    dv = idx_v[1, pl.ds(k * CH, CH)]
                    sd.append(
                        pltpu.async_copy(
                            rows_v.at[pl.ds(k * CH, CH)],
                            acc_s.at[dv],
                            ssem,
                            add=True,
                        )
                    )
                for d in sd:
                    d.wait()
                return c

            lax.fori_loop(0, n_chunks // G, body, 0)

        if feature_split:
            pl.when(cid == 0)(lambda: run(tables[0]))
            pl.when(cid == 1)(lambda: run(tables[1]))
        else:
            run(tables[0])
        plsc.subcore_barrier()
        pltpu.sync_copy(acc_s.at[pl.ds(r0, rpt)], out_hbm.at[cid, pl.ds(r0, rpt)])

    return agg_k


def _dot(a, b):
    return jax.lax.dot_general(
        a, b, (((1,), (0,)), ((), ())),
        preferred_element_type=jnp.float32,
        precision=jax.lax.Precision.HIGHEST,
    )


def _prep1(x, W1a, W1b, degh, BLK=1000):
    N, K = x.shape

    def body(x_ref, wa_ref, wb_ref, dg_ref, lo_ref, hi_ref, dv_ref):
        deg = dg_ref[0, :, 0] + dg_ref[1, :, 0] + 1.0
        dinv = lax.rsqrt(deg)
        xs = x_ref[...] * dinv[:, None]
        lo_ref[...] = _dot(xs, wa_ref[...])
        hi_ref[...] = _dot(xs, wb_ref[...])
        dv_ref[...] = jnp.broadcast_to(dinv[:, None], (BLK, D))

    return pl.pallas_call(
        body,
        grid=(N // BLK,),
        in_specs=[
            pl.BlockSpec((BLK, K), lambda i: (i, 0)),
            pl.BlockSpec((K, D), lambda i: (0, 0)),
            pl.BlockSpec((K, D), lambda i: (0, 0)),
            pl.BlockSpec((NC, BLK, D), lambda i: (0, i, 0)),
        ],
        out_specs=[pl.BlockSpec((BLK, D), lambda i: (i, 0))] * 3,
        out_shape=[jax.ShapeDtypeStruct((N, D), jnp.float32)] * 3,
    )(x, W1a, W1b, degh)


def _mid(agg1, hs_lo, hs_hi, dinvb, b1a, b1b, W2a, W2b, BLK=1000):
    N = hs_lo.shape[0]

    def body(ag_ref, lo_ref, hi_ref, dv_ref, ba_ref, bb_ref, wa_ref, wb_ref, o_ref):
        dinv = dv_ref[...]
        ta = jax.nn.relu(dinv * (ag_ref[0] + lo_ref[...]) + ba_ref[...]) * dinv
        tb = jax.nn.relu(dinv * (ag_ref[1] + hi_ref[...]) + bb_ref[...]) * dinv
        o_ref[...] = _dot(ta, wa_ref[...]) + _dot(tb, wb_ref[...])

    return pl.pallas_call(
        body,
        grid=(N // BLK,),
        in_specs=[
            pl.BlockSpec((NC, BLK, D), lambda i: (0, i, 0)),
            pl.BlockSpec((BLK, D), lambda i: (i, 0)),
            pl.BlockSpec((BLK, D), lambda i: (i, 0)),
            pl.BlockSpec((BLK, D), lambda i: (i, 0)),
            pl.BlockSpec((1, D), lambda i: (0, 0)),
            pl.BlockSpec((1, D), lambda i: (0, 0)),
            pl.BlockSpec((D, D), lambda i: (0, 0)),
            pl.BlockSpec((D, D), lambda i: (0, 0)),
        ],
        out_specs=pl.BlockSpec((BLK, D), lambda i: (i, 0)),
        out_shape=jax.ShapeDtypeStruct((N, D), jnp.float32),
    )(agg1, hs_lo, hs_hi, dinvb, b1a, b1b, W2a, W2b)


def _fin(agg2, hs2, dinvb, b2r, Wlin, blinr, BLK=1000):
    N = hs2.shape[0]

    def body(ag_ref, h_ref, dv_ref, b_ref, w_ref, bl_ref, o_ref):
        dinv = dv_ref[...]
        agg = ag_ref[0] + ag_ref[1]
        t = jax.nn.relu(dinv * (agg + h_ref[...]) + b_ref[...])
        o_ref[...] = _dot(t, w_ref[...]) + bl_ref[...]

    return pl.pallas_call(
        body,
        grid=(N // BLK,),
        in_specs=[
            pl.BlockSpec((NC, BLK, D), lambda i: (0, i, 0)),
            pl.BlockSpec((BLK, D), lambda i: (i, 0)),
            pl.BlockSpec((BLK, D), lambda i: (i, 0)),
            pl.BlockSpec((1, D), lambda i: (0, 0)),
            pl.BlockSpec((D, D), lambda i: (0, 0)),
            pl.BlockSpec((1, D), lambda i: (0, 0)),
        ],
        out_specs=pl.BlockSpec((BLK, D), lambda i: (i, 0)),
        out_shape=jax.ShapeDtypeStruct((N, D), jnp.float32),
    )(agg2, hs2, dinvb, b2r, Wlin, blinr)


def kernel(x, edge_index, W1, b1, W2, b2, Wlin, blin):
    N = x.shape[0]
    E = edge_index.shape[1]
    N_pad = -(-N // (NS * 8)) * (NS * 8)
    ei = edge_index.astype(jnp.int32)
    src, dst = ei[0], ei[1]
    zeros = jnp.zeros((N_pad, D), jnp.float32)
    ones = jnp.ones((CH, D), jnp.float32)

    def pack(T, G):
        g_len = G * CH
        ng = E // (T * g_len)
        return jnp.stack(
            [src.reshape(T, ng, g_len), dst.reshape(T, ng, g_len)], axis=2
        )

    degh = _deg_call(E, N_pad)(dst, ones, zeros)
    hs_lo, hs_hi, dinvb = _prep1(x, W1[:, :D], W1[:, D:], degh)
    agg1 = _agg_call(E, N_pad, True)(hs_lo, hs_hi, pack(NS, 10), zeros)
    hs2 = _mid(
        agg1, hs_lo, hs_hi, dinvb,
        b1[:D].reshape(1, D), b1[D:].reshape(1, D), W2[:D], W2[D:],
    )
    agg2 = _agg_call(E, N_pad, False)(hs2, pack(NC * NS, 5), zeros)
    return _fin(agg2, hs2, dinvb, b2.reshape(1, D), Wlin, blin.reshape(1, D))
